# baseline (speedup 1.0000x reference)
import functools

import jax
import jax.numpy as jnp
from jax import lax
from jax.experimental import pallas as pl
from jax.experimental.pallas import tpu as pltpu
from jax.experimental.pallas import tpu_sc as plsc

F32 = jnp.float32
I32 = jnp.int32
NC, NS = 2, 16
CH = 128
BLK = 512

@functools.cache
def _mesh():
    return plsc.VectorSubcoreMesh(
        core_axis_name="c", subcore_axis_name="s",
        num_cores=NC, num_subcores=NS)


def _mm(x_pad, w, npad, d):
    grid = npad // BLK

    def body(x_ref, w_ref, o_ref):
        o_ref[...] = jnp.dot(x_ref[...], w_ref[...],
                             preferred_element_type=F32)

    return pl.pallas_call(
        body,
        grid=(grid,),
        in_specs=[pl.BlockSpec((BLK, d), lambda i: (i, 0)),
                  pl.BlockSpec((d, d), lambda i: (0, 0))],
        out_specs=pl.BlockSpec((BLK, d), lambda i: (i, 0)),
        out_shape=jax.ShapeDtypeStruct((npad, d), F32),
    )(x_pad, w)


def _sc_deg(dst_deg, zeros_h, ones_h, npad, nch2, h):
    seg = npad // NS

    @functools.partial(
        pl.kernel,
        out_type=jax.ShapeDtypeStruct((NC, npad, h), F32),
        mesh=_mesh(),
        scratch_types=[
            pltpu.VMEM((nch2, CH), I32),
            pltpu.VMEM((CH, h), F32),
            pltpu.VMEM_SHARED((npad, h), F32),
        ],
    )
    def k(dst_hbm, zeros_hbm, ones_hbm, out_hbm, idx_v, ones_v, deg_sp):
        c = lax.axis_index("c")
        s = lax.axis_index("s")
        wid = c * NS + s
        pltpu.sync_copy(dst_hbm.at[wid], idx_v)
        pltpu.sync_copy(ones_hbm, ones_v)
        pltpu.sync_copy(zeros_hbm.at[pl.ds(s * seg, seg)],
                        deg_sp.at[pl.ds(s * seg, seg)])
        plsc.subcore_barrier()

        def body(j, carry):
            pltpu.sync_copy(ones_v, deg_sp.at[idx_v.at[j]], add=True)
            return carry

        lax.fori_loop(0, nch2, body, 0)
        plsc.subcore_barrier()

        @pl.when(c == 0)
        def _():
            pltpu.sync_copy(deg_sp.at[pl.ds(s * seg, seg)],
                            out_hbm.at[0].at[pl.ds(s * seg, seg)])

        @pl.when(c == 1)
        def _():
            pltpu.sync_copy(deg_sp.at[pl.ds(s * seg, seg)],
                            out_hbm.at[1].at[pl.ds(s * seg, seg)])

    return k(dst_deg, zeros_h, ones_h)


NBUF = 2


def _sc_agg(y, pak_t, npad, h, nch):
    seg = npad // NS
    assert nch % NBUF == 0

    @functools.partial(
        pl.kernel,
        out_type=jax.ShapeDtypeStruct((NC, npad, h), F32),
        mesh=_mesh(),
        scratch_types=(
            [pltpu.VMEM((nch, CH), I32)]
            + [pltpu.VMEM((2, CH), I32)] * NBUF
            + [pltpu.VMEM((CH, h), F32)] * NBUF
            + [pltpu.VMEM_SHARED((npad, h), F32)]
            + [pltpu.SemaphoreType.DMA] * (2 * NBUF)
        ),
    )
    def k(y_hbm, pak_hbm, out_hbm, *scr):
        pak_v = scr[0]
        idxs = scr[1:1 + NBUF]
        bufs = scr[1 + NBUF:1 + 2 * NBUF]
        acc_sp = scr[1 + 2 * NBUF]
        gsems = scr[2 + 2 * NBUF:2 + 3 * NBUF]
        ssems = scr[2 + 3 * NBUF:2 + 4 * NBUF]
        c = lax.axis_index("c")
        s = lax.axis_index("s")
        pltpu.sync_copy(pak_hbm.at[s], pak_v)

        def unpack(j, slot):
            for i in range(CH // 16):
                v = pak_v[j, pl.ds(i * 16, 16)]
                idxs[slot][0, pl.ds(i * 16, 16)] = jnp.bitwise_and(
                    v, jnp.int32(0xFFFF))
                idxs[slot][1, pl.ds(i * 16, 16)] = lax.shift_right_logical(
                    v, jnp.int32(16))

        def run(yh, oh):
            pltpu.sync_copy(yh.at[pl.ds(s * seg, seg)],
                            acc_sp.at[pl.ds(s * seg, seg)])
            plsc.subcore_barrier()

            for b in range(NBUF):
                unpack(b, b)
                pltpu.async_copy(yh.at[idxs[b].at[0]], bufs[b], gsems[b])

            def round_(k0, last):
                dscs = []
                for b in range(NBUF):
                    pltpu.make_async_copy(
                        yh.at[pl.ds(0, CH)], bufs[b], gsems[b]).wait()
                    dscs.append(pltpu.async_copy(
                        bufs[b], acc_sp.at[idxs[b].at[1]], ssems[b],
                        add=True))
                for b in range(NBUF):
                    dscs[b].wait()
                    if not last:
                        unpack(k0 + b + NBUF, b)
                        pltpu.async_copy(
                            yh.at[idxs[b].at[0]], bufs[b], gsems[b])

            def body(j, carry):
                round_(j * NBUF, last=False)
                return carry

            lax.fori_loop(0, nch // NBUF - 1, body, 0)
            round_(nch - NBUF, last=True)
            plsc.subcore_barrier()
            pltpu.sync_copy(acc_sp.at[pl.ds(s * seg, seg)],
                            oh.at[pl.ds(s * seg, seg)])

        @pl.when(c == 0)
        def _():
            run(y_hbm.at[0], out_hbm.at[0])

        @pl.when(c == 1)
        def _():
            run(y_hbm.at[1], out_hbm.at[1])

    return k(y, pak_t)


def _bcast(col, n):
    return lax.broadcast_in_dim(col, (col.shape[0], n), (0, 1))


def _scale_split(degp, xw, npad, d):
    grid = npad // BLK
    h = d // 2

    def body(deg_ref, xw_ref, y_ref, dinv_ref):
        deg = deg_ref[0][:, 0:1] + deg_ref[1][:, 0:1]
        dinv = lax.rsqrt(deg + 1.0)
        dv = _bcast(dinv, h)
        xw_v = xw_ref[...]
        y_ref[0] = xw_v[:, :h] * dv
        y_ref[1] = xw_v[:, h:] * dv
        dinv_ref[...] = _bcast(dinv, 8)

    return pl.pallas_call(
        body,
        grid=(grid,),
        in_specs=[pl.BlockSpec((NC, BLK, h), lambda i: (0, i, 0)),
                  pl.BlockSpec((BLK, d), lambda i: (i, 0))],
        out_specs=[pl.BlockSpec((NC, BLK, h), lambda i: (0, i, 0)),
                   pl.BlockSpec((BLK, 8), lambda i: (i, 0))],
        out_shape=[jax.ShapeDtypeStruct((NC, npad, h), F32),
                   jax.ShapeDtypeStruct((npad, 8), F32)],
    )(degp, xw)


def _layer2(acc1, dinv8, b1r, w2, npad, d):
    grid = npad // BLK
    h = d // 2

    def body(a_ref, dinv_ref, b1_ref, w2_ref, o_ref):
        dv = _bcast(dinv_ref[...][:, 0:1], h)
        hid = jnp.concatenate([a_ref[0] * dv, a_ref[1] * dv], axis=1)
        hid = jnp.maximum(hid + b1_ref[...], 0.0)
        y2 = jnp.dot(hid, w2_ref[...], preferred_element_type=F32)
        o_ref[0] = y2[:, :h] * dv
        o_ref[1] = y2[:, h:] * dv

    return pl.pallas_call(
        body,
        grid=(grid,),
        in_specs=[pl.BlockSpec((NC, BLK, h), lambda i: (0, i, 0)),
                  pl.BlockSpec((BLK, 8), lambda i: (i, 0)),
                  pl.BlockSpec((1, d), lambda i: (0, 0)),
                  pl.BlockSpec((d, d), lambda i: (0, 0))],
        out_specs=pl.BlockSpec((NC, BLK, h), lambda i: (0, i, 0)),
        out_shape=jax.ShapeDtypeStruct((NC, npad, h), F32),
    )(acc1, dinv8, b1r, w2)


def _head(acc2, dinv8, b2r, batch8, wl1, bl1r, wl2, bl2r, wl3, bl3r,
          npad, d, g):
    grid = npad // BLK
    h = d // 2
    d1 = wl1.shape[1]
    d2 = wl2.shape[1]

    def body(a_ref, dinv_ref, b2_ref, batch_ref, wl1_ref, bl1_ref,
             wl2_ref, bl2_ref, wl3_ref, bl3_ref, o_ref, sums_ref, cnt_ref):
        i = pl.program_id(0)

        @pl.when(i == 0)
        def _():
            sums_ref[...] = jnp.zeros_like(sums_ref)
            cnt_ref[...] = jnp.zeros_like(cnt_ref)

        dv = _bcast(dinv_ref[...][:, 0:1], h)
        h2 = jnp.concatenate([a_ref[0] * dv, a_ref[1] * dv], axis=1)
        h2 = h2 + b2_ref[...]
        bb = _bcast(batch_ref[...][:, 0:1], g)
        gids = lax.broadcasted_iota(I32, (BLK, g), 1)
        m = (bb == gids).astype(F32)
        sums_ref[...] += lax.dot_general(
            m, h2, (((0,), (0,)), ((), ())), preferred_element_type=F32)
        cnt_ref[...] += lax.dot_general(
            m, jnp.ones((BLK, d), F32), (((0,), (0,)), ((), ())),
            preferred_element_type=F32)

        @pl.when(i == grid - 1)
        def _():
            gm = sums_ref[...] / jnp.maximum(cnt_ref[...], 1.0)
            r = jnp.dot(gm, wl1_ref[...], preferred_element_type=F32)
            r = jnp.maximum(r + bl1_ref[...], 0.0)
            r = jnp.dot(r, wl2_ref[...], preferred_element_type=F32)
            r = jnp.maximum(r + bl2_ref[...], 0.0)
            o_ref[...] = (jnp.dot(r, wl3_ref[...], preferred_element_type=F32)
                          + bl3_ref[...])

    return pl.pallas_call(
        body,
        grid=(grid,),
        in_specs=[pl.BlockSpec((NC, BLK, h), lambda i: (0, i, 0)),
                  pl.BlockSpec((BLK, 8), lambda i: (i, 0)),
                  pl.BlockSpec((1, d), lambda i: (0, 0)),
                  pl.BlockSpec((BLK, 8), lambda i: (i, 0)),
                  pl.BlockSpec((d, d1), lambda i: (0, 0)),
                  pl.BlockSpec((1, d1), lambda i: (0, 0)),
                  pl.BlockSpec((d1, d2), lambda i: (0, 0)),
                  pl.BlockSpec((1, d2), lambda i: (0, 0)),
                  pl.BlockSpec((d2, 1), lambda i: (0, 0)),
                  pl.BlockSpec((1, 1), lambda i: (0, 0))],
        out_specs=pl.BlockSpec((g, 1), lambda i: (0, 0)),
        out_shape=jax.ShapeDtypeStruct((g, 1), F32),
        scratch_shapes=[pltpu.VMEM((g, d), F32), pltpu.VMEM((g, d), F32)],
    )(acc2, dinv8, b2r, batch8, wl1, bl1r, wl2, bl2r, wl3, bl3r)


def kernel(x, edge_index, batch, W1, b1, W2, b2,
           Wl1, bl1, Wl2, bl2, Wl3, bl3):
    N, D = x.shape
    E = edge_index.shape[1]
    G = 64
    NPAD = 10240
    EPAD = 163840
    H = D // 2
    EPT = EPAD // NS
    NCH = EPT // CH
    EPT2 = EPAD // (NC * NS)
    NCH2 = EPT2 // CH

    x_pad = jnp.concatenate([x, jnp.zeros((NPAD - N, D), F32)], axis=0)
    src = jnp.concatenate(
        [edge_index[0], jnp.zeros((EPAD - E,), I32)])
    dst = jnp.concatenate(
        [edge_index[1], jnp.full((EPAD - E,), NPAD - 8, I32)])
    pak_t = jnp.bitwise_or(jnp.left_shift(dst, 16), src).reshape(NS, NCH, CH)
    dst_deg = dst.reshape(NC * NS, NCH2, CH)
    zerosh = jnp.zeros((NPAD, H), F32)
    onesh = jnp.ones((CH, H), F32)
    batch8 = jnp.broadcast_to(
        jnp.concatenate([batch, jnp.full((NPAD - N,), G, I32)])[:, None],
        (NPAD, 8))
    b1r = b1.reshape(1, D)
    b2r = b2.reshape(1, D)
    bl1r = bl1.reshape(1, -1)
    bl2r = bl2.reshape(1, -1)
    bl3r = bl3.reshape(1, -1)

    xw1 = _mm(x_pad, W1, NPAD, D)
    degp = _sc_deg(dst_deg, zerosh, onesh, NPAD, NCH2, H)
    y1, dinv8 = _scale_split(degp, xw1, NPAD, D)
    acc1 = _sc_agg(y1, pak_t, NPAD, H, NCH)
    y2 = _layer2(acc1, dinv8, b1r, W2, NPAD, D)
    acc2 = _sc_agg(y2, pak_t, NPAD, H, NCH)
    return _head(acc2, dinv8, b2r, batch8, Wl1, bl1r, Wl2, bl2r, Wl3, bl3r,
                 NPAD, D, G)


# speedup vs baseline: 1.0880x; 1.0880x over previous
import functools

import jax
import jax.numpy as jnp
from jax import lax
from jax.experimental import pallas as pl
from jax.experimental.pallas import tpu as pltpu
from jax.experimental.pallas import tpu_sc as plsc

F32 = jnp.float32
I32 = jnp.int32
NC, NS = 2, 16
CH = 128
BLK = 512

@functools.cache
def _mesh():
    return plsc.VectorSubcoreMesh(
        core_axis_name="c", subcore_axis_name="s",
        num_cores=NC, num_subcores=NS)


def _mm(x_pad, w, npad, d):
    grid = npad // BLK

    def body(x_ref, w_ref, o_ref):
        o_ref[...] = jnp.dot(x_ref[...], w_ref[...],
                             preferred_element_type=F32)

    return pl.pallas_call(
        body,
        grid=(grid,),
        in_specs=[pl.BlockSpec((BLK, d), lambda i: (i, 0)),
                  pl.BlockSpec((d, d), lambda i: (0, 0))],
        out_specs=pl.BlockSpec((BLK, d), lambda i: (i, 0)),
        out_shape=jax.ShapeDtypeStruct((npad, d), F32),
    )(x_pad, w)


def _sc_deg(dst_deg, zeros_h, ones_h, npad, nch2, h):
    seg = npad // NS

    @functools.partial(
        pl.kernel,
        out_type=jax.ShapeDtypeStruct((NC, npad, h), F32),
        mesh=_mesh(),
        scratch_types=[
            pltpu.VMEM((nch2, CH), I32),
            pltpu.VMEM((CH, h), F32),
            pltpu.VMEM_SHARED((npad, h), F32),
        ],
    )
    def k(dst_hbm, zeros_hbm, ones_hbm, out_hbm, idx_v, ones_v, deg_sp):
        c = lax.axis_index("c")
        s = lax.axis_index("s")
        wid = c * NS + s
        pltpu.sync_copy(dst_hbm.at[wid], idx_v)
        pltpu.sync_copy(ones_hbm, ones_v)
        pltpu.sync_copy(zeros_hbm.at[pl.ds(s * seg, seg)],
                        deg_sp.at[pl.ds(s * seg, seg)])
        plsc.subcore_barrier()

        def body(j, carry):
            pltpu.sync_copy(ones_v, deg_sp.at[idx_v.at[j]], add=True)
            return carry

        lax.fori_loop(0, nch2, body, 0)
        plsc.subcore_barrier()

        @pl.when(c == 0)
        def _():
            pltpu.sync_copy(deg_sp.at[pl.ds(s * seg, seg)],
                            out_hbm.at[0].at[pl.ds(s * seg, seg)])

        @pl.when(c == 1)
        def _():
            pltpu.sync_copy(deg_sp.at[pl.ds(s * seg, seg)],
                            out_hbm.at[1].at[pl.ds(s * seg, seg)])

    return k(dst_deg, zeros_h, ones_h)


NBUF = 4
CHD = 64


def _sc_agg(y, pak_t, npad, h, nch):
    seg = npad // NS
    assert nch % NBUF == 0

    @functools.partial(
        pl.kernel,
        out_type=jax.ShapeDtypeStruct((NC, npad, h), F32),
        mesh=_mesh(),
        scratch_types=(
            [pltpu.VMEM((nch, CH), I32)]
            + [pltpu.VMEM((2, CHD), I32)] * NBUF
            + [pltpu.VMEM((CHD, h), F32)] * NBUF
            + [pltpu.VMEM_SHARED((npad, h), F32)]
            + [pltpu.SemaphoreType.DMA] * NBUF
        ),
    )
    def k(y_hbm, pak_hbm, out_hbm, *scr):
        pak_v = scr[0]
        idxs = scr[1:1 + NBUF]
        bufs = scr[1 + NBUF:1 + 2 * NBUF]
        acc_sp = scr[1 + 2 * NBUF]
        gsems = scr[2 + 2 * NBUF:2 + 3 * NBUF]
        c = lax.axis_index("c")
        s = lax.axis_index("s")
        pltpu.sync_copy(pak_hbm.at[s], pak_v)

        def unpack(row, half, slot):
            for i in range(CHD // 16):
                v = pak_v[row, pl.ds(half * CHD + i * 16, 16)]
                idxs[slot][0, pl.ds(i * 16, 16)] = jnp.bitwise_and(
                    v, jnp.int32(0xFFFF))
                idxs[slot][1, pl.ds(i * 16, 16)] = lax.shift_right_logical(
                    v, jnp.int32(16))

        def run(yh, oh):
            pltpu.sync_copy(yh.at[pl.ds(s * seg, seg)],
                            acc_sp.at[pl.ds(s * seg, seg)])
            plsc.subcore_barrier()

            for b in range(NBUF):
                unpack(b // 2, b % 2, b)
                pltpu.async_copy(yh.at[idxs[b].at[0]], bufs[b], gsems[b])

            def round_(r0, last):
                for b in range(NBUF):
                    pltpu.make_async_copy(
                        yh.at[pl.ds(0, CHD)], bufs[b], gsems[b]).wait()
                    pltpu.sync_copy(bufs[b], acc_sp.at[idxs[b].at[1]],
                                    add=True)
                    if not last:
                        unpack(r0 + 2 + b // 2, b % 2, b)
                        pltpu.async_copy(
                            yh.at[idxs[b].at[0]], bufs[b], gsems[b])

            def body(j, carry):
                round_(j * 2, last=False)
                return carry

            lax.fori_loop(0, nch // 2 - 1, body, 0)
            round_(nch - 2, last=True)
            plsc.subcore_barrier()
            pltpu.sync_copy(acc_sp.at[pl.ds(s * seg, seg)],
                            oh.at[pl.ds(s * seg, seg)])

        @pl.when(c == 0)
        def _():
            run(y_hbm.at[0], out_hbm.at[0])

        @pl.when(c == 1)
        def _():
            run(y_hbm.at[1], out_hbm.at[1])

    return k(y, pak_t)


def _bcast(col, n):
    return lax.broadcast_in_dim(col, (col.shape[0], n), (0, 1))


def _scale_split(degp, xw, npad, d):
    grid = npad // BLK
    h = d // 2

    def body(deg_ref, xw_ref, y_ref, dinv_ref):
        deg = deg_ref[0][:, 0:1] + deg_ref[1][:, 0:1]
        dinv = lax.rsqrt(deg + 1.0)
        dv = _bcast(dinv, h)
        xw_v = xw_ref[...]
        y_ref[0] = xw_v[:, :h] * dv
        y_ref[1] = xw_v[:, h:] * dv
        dinv_ref[...] = _bcast(dinv, 8)

    return pl.pallas_call(
        body,
        grid=(grid,),
        in_specs=[pl.BlockSpec((NC, BLK, h), lambda i: (0, i, 0)),
                  pl.BlockSpec((BLK, d), lambda i: (i, 0))],
        out_specs=[pl.BlockSpec((NC, BLK, h), lambda i: (0, i, 0)),
                   pl.BlockSpec((BLK, 8), lambda i: (i, 0))],
        out_shape=[jax.ShapeDtypeStruct((NC, npad, h), F32),
                   jax.ShapeDtypeStruct((npad, 8), F32)],
    )(degp, xw)


def _layer2(acc1, dinv8, b1r, w2, npad, d):
    grid = npad // BLK
    h = d // 2

    def body(a_ref, dinv_ref, b1_ref, w2_ref, o_ref):
        dv = _bcast(dinv_ref[...][:, 0:1], h)
        hid = jnp.concatenate([a_ref[0] * dv, a_ref[1] * dv], axis=1)
        hid = jnp.maximum(hid + b1_ref[...], 0.0)
        y2 = jnp.dot(hid, w2_ref[...], preferred_element_type=F32)
        o_ref[0] = y2[:, :h] * dv
        o_ref[1] = y2[:, h:] * dv

    return pl.pallas_call(
        body,
        grid=(grid,),
        in_specs=[pl.BlockSpec((NC, BLK, h), lambda i: (0, i, 0)),
                  pl.BlockSpec((BLK, 8), lambda i: (i, 0)),
                  pl.BlockSpec((1, d), lambda i: (0, 0)),
                  pl.BlockSpec((d, d), lambda i: (0, 0))],
        out_specs=pl.BlockSpec((NC, BLK, h), lambda i: (0, i, 0)),
        out_shape=jax.ShapeDtypeStruct((NC, npad, h), F32),
    )(acc1, dinv8, b1r, w2)


def _head(acc2, dinv8, b2r, batch8, wl1, bl1r, wl2, bl2r, wl3, bl3r,
          npad, d, g):
    grid = npad // BLK
    h = d // 2
    d1 = wl1.shape[1]
    d2 = wl2.shape[1]

    def body(a_ref, dinv_ref, b2_ref, batch_ref, wl1_ref, bl1_ref,
             wl2_ref, bl2_ref, wl3_ref, bl3_ref, o_ref, sums_ref, cnt_ref):
        i = pl.program_id(0)

        @pl.when(i == 0)
        def _():
            sums_ref[...] = jnp.zeros_like(sums_ref)
            cnt_ref[...] = jnp.zeros_like(cnt_ref)

        dv = _bcast(dinv_ref[...][:, 0:1], h)
        h2 = jnp.concatenate([a_ref[0] * dv, a_ref[1] * dv], axis=1)
        h2 = h2 + b2_ref[...]
        bb = _bcast(batch_ref[...][:, 0:1], g)
        gids = lax.broadcasted_iota(I32, (BLK, g), 1)
        m = (bb == gids).astype(F32)
        sums_ref[...] += lax.dot_general(
            m, h2, (((0,), (0,)), ((), ())), preferred_element_type=F32)
        cnt_ref[...] += lax.dot_general(
            m, jnp.ones((BLK, d), F32), (((0,), (0,)), ((), ())),
            preferred_element_type=F32)

        @pl.when(i == grid - 1)
        def _():
            gm = sums_ref[...] / jnp.maximum(cnt_ref[...], 1.0)
            r = jnp.dot(gm, wl1_ref[...], preferred_element_type=F32)
            r = jnp.maximum(r + bl1_ref[...], 0.0)
            r = jnp.dot(r, wl2_ref[...], preferred_element_type=F32)
            r = jnp.maximum(r + bl2_ref[...], 0.0)
            o_ref[...] = (jnp.dot(r, wl3_ref[...], preferred_element_type=F32)
                          + bl3_ref[...])

    return pl.pallas_call(
        body,
        grid=(grid,),
        in_specs=[pl.BlockSpec((NC, BLK, h), lambda i: (0, i, 0)),
                  pl.BlockSpec((BLK, 8), lambda i: (i, 0)),
                  pl.BlockSpec((1, d), lambda i: (0, 0)),
                  pl.BlockSpec((BLK, 8), lambda i: (i, 0)),
                  pl.BlockSpec((d, d1), lambda i: (0, 0)),
                  pl.BlockSpec((1, d1), lambda i: (0, 0)),
                  pl.BlockSpec((d1, d2), lambda i: (0, 0)),
                  pl.BlockSpec((1, d2), lambda i: (0, 0)),
                  pl.BlockSpec((d2, 1), lambda i: (0, 0)),
                  pl.BlockSpec((1, 1), lambda i: (0, 0))],
        out_specs=pl.BlockSpec((g, 1), lambda i: (0, 0)),
        out_shape=jax.ShapeDtypeStruct((g, 1), F32),
        scratch_shapes=[pltpu.VMEM((g, d), F32), pltpu.VMEM((g, d), F32)],
    )(acc2, dinv8, b2r, batch8, wl1, bl1r, wl2, bl2r, wl3, bl3r)


def kernel(x, edge_index, batch, W1, b1, W2, b2,
           Wl1, bl1, Wl2, bl2, Wl3, bl3):
    N, D = x.shape
    E = edge_index.shape[1]
    G = 64
    NPAD = 10240
    EPAD = 163840
    H = D // 2
    EPT = EPAD // NS
    NCH = EPT // CH
    EPT2 = EPAD // (NC * NS)
    NCH2 = EPT2 // CH

    x_pad = jnp.concatenate([x, jnp.zeros((NPAD - N, D), F32)], axis=0)
    src = jnp.concatenate(
        [edge_index[0], jnp.zeros((EPAD - E,), I32)])
    dst = jnp.concatenate(
        [edge_index[1], jnp.full((EPAD - E,), NPAD - 8, I32)])
    pak_t = jnp.bitwise_or(jnp.left_shift(dst, 16), src).reshape(NS, NCH, CH)
    dst_deg = dst.reshape(NC * NS, NCH2, CH)
    zerosh = jnp.zeros((NPAD, H), F32)
    onesh = jnp.ones((CH, H), F32)
    batch8 = jnp.broadcast_to(
        jnp.concatenate([batch, jnp.full((NPAD - N,), G, I32)])[:, None],
        (NPAD, 8))
    b1r = b1.reshape(1, D)
    b2r = b2.reshape(1, D)
    bl1r = bl1.reshape(1, -1)
    bl2r = bl2.reshape(1, -1)
    bl3r = bl3.reshape(1, -1)

    xw1 = _mm(x_pad, W1, NPAD, D)
    degp = _sc_deg(dst_deg, zerosh, onesh, NPAD, NCH2, H)
    y1, dinv8 = _scale_split(degp, xw1, NPAD, D)
    acc1 = _sc_agg(y1, pak_t, NPAD, H, NCH)
    y2 = _layer2(acc1, dinv8, b1r, W2, NPAD, D)
    acc2 = _sc_agg(y2, pak_t, NPAD, H, NCH)
    return _head(acc2, dinv8, b2r, batch8, Wl1, bl1r, Wl2, bl2r, Wl3, bl3r,
                 NPAD, D, G)
